# baseline (speedup 1.0000x reference)
import functools

import jax
import jax.numpy as jnp
import numpy as np
from jax import lax
from jax.experimental import pallas as pl
from jax.experimental.pallas import tpu as pltpu
from jax.experimental.pallas import tpu_sc as plsc

B = 64
N = 3 * 224 * 224
W = 10752
NWIN = N // W
VPW = W // 16
NBINS = 1 << 15
NC, NS = 2, 16
SAMPLES_PER_TILE = B // (NC * NS)

_PERM = (
    19, 54, 30, 7, 6, 35, 23, 58, 16, 21, 61, 38, 3, 26, 32, 37,
    56, 51, 2, 63, 52, 20, 44, 47, 50, 42, 62, 53, 0, 8, 22, 13,
    29, 34, 18, 24, 1, 48, 5, 45, 49, 33, 55, 60, 57, 27, 10, 15,
    40, 17, 59, 36, 28, 46, 9, 4, 12, 14, 31, 41, 25, 43, 39, 11,
)


def _mesh():
  return plsc.VectorSubcoreMesh(
      core_axis_name="c", subcore_axis_name="s", num_cores=NC, num_subcores=NS)


def _sort_body(x_hbm, sums, s1k, s1i, offs2,
               hist1, hist2, xbuf, kbuf, ibuf, dbuf, v16, sem0, sem1):
  wid = lax.axis_index("c") * NS + lax.axis_index("s")
  lanes = lax.iota(jnp.int32, 16)

  for sl in range(SAMPLES_PER_TILE):
    samp = wid * SAMPLES_PER_TILE + sl
    base = samp * N

    def zero_body(i, carry):
      z = jnp.zeros((16,), jnp.int32)
      hist1[pl.ds(i * 16, 16)] = z
      hist2[pl.ds(i * 16, 16)] = z
      return carry
    lax.fori_loop(0, NBINS // 16, zero_body, 0)

    def histo_win(w, vsum):
      pltpu.sync_copy(x_hbm.at[pl.ds(base + w * W, W)], xbuf)
      def body(i, vs):
        h = xbuf[pl.ds(i * 16, 16)] + 0.5
        k = lax.bitcast_convert_type(h, jnp.int32)
        d1 = k & (NBINS - 1)
        d2 = lax.shift_right_logical(k, 15)
        c1, m1 = plsc.scan_count(d1)
        plsc.addupdate_scatter(hist1, [d1], c1, mask=m1)
        c2, m2 = plsc.scan_count(d2)
        plsc.addupdate_scatter(hist2, [d2], c2, mask=m2)
        return vs + h
      return lax.fori_loop(0, VPW, body, vsum)
    vsum = lax.fori_loop(0, NWIN, histo_win, jnp.zeros((16,), jnp.float32))
    v16[...] = vsum
    pltpu.sync_copy(v16, sums.at[pl.ds(samp * 16, 16)])

    def prefix(hist):
      def body(i, carry):
        v = hist[pl.ds(i * 16, 16)]
        cs = plsc.cumsum(v)
        hist[pl.ds(i * 16, 16)] = carry + (cs - v)
        return carry + jnp.sum(v)
      lax.fori_loop(0, NBINS // 16, body,
                    jnp.full((16,), base, jnp.int32))
    prefix(hist1)
    prefix(hist2)
    pltpu.sync_copy(hist2, offs2.at[pl.ds(samp * NBINS, NBINS)])

    def low_win(w, carry):
      wbase = base + w * W
      pltpu.sync_copy(x_hbm.at[pl.ds(wbase, W)], xbuf)
      def body(i, c):
        h = xbuf[pl.ds(i * 16, 16)] + 0.5
        k = lax.bitcast_convert_type(h, jnp.int32)
        d1 = k & (NBINS - 1)
        cnt, mlast = plsc.scan_count(d1)
        old = plsc.load_gather(hist1, [d1])
        dbuf[pl.ds(i * 16, 16)] = old + cnt - 1
        kbuf[pl.ds(i * 16, 16)] = h
        ibuf[pl.ds(i * 16, 16)] = (wbase + i * 16) + lanes
        plsc.addupdate_scatter(hist1, [d1], cnt, mask=mlast)
        return c
      lax.fori_loop(0, VPW, body, 0)
      cp0 = pltpu.async_copy(kbuf, s1k.at[dbuf], sem0)
      cp1 = pltpu.async_copy(ibuf, s1i.at[dbuf], sem1)
      cp0.wait()
      cp1.wait()
      return carry
    lax.fori_loop(0, NWIN, low_win, 0)


def _highpass_body(s1k, s1i, offs2, s2k, s2i,
                   hist2, xbuf, ibuf, dbuf, sem0, sem1):
  wid = lax.axis_index("c") * NS + lax.axis_index("s")
  for sl in range(SAMPLES_PER_TILE):
    samp = wid * SAMPLES_PER_TILE + sl
    base = samp * N
    pltpu.sync_copy(offs2.at[pl.ds(samp * NBINS, NBINS)], hist2)
    def high_win(w, carry):
      wbase = base + w * W
      pltpu.sync_copy(s1k.at[pl.ds(wbase, W)], xbuf)
      pltpu.sync_copy(s1i.at[pl.ds(wbase, W)], ibuf)
      def body(i, c):
        h = xbuf[pl.ds(i * 16, 16)]
        k = lax.bitcast_convert_type(h, jnp.int32)
        d2 = lax.shift_right_logical(k, 15)
        cnt, mlast = plsc.scan_count(d2)
        old = plsc.load_gather(hist2, [d2])
        dbuf[pl.ds(i * 16, 16)] = old + cnt - 1
        plsc.addupdate_scatter(hist2, [d2], cnt, mask=mlast)
        return c
      lax.fori_loop(0, VPW, body, 0)
      cp0 = pltpu.async_copy(xbuf, s2k.at[dbuf], sem0)
      cp1 = pltpu.async_copy(ibuf, s2i.at[dbuf], sem1)
      cp0.wait()
      cp1.wait()
      return carry
    lax.fori_loop(0, NWIN, high_win, 0)


def _match_body(s2k, s2i, sums, perm, out, xbuf, ibuf, pvec, v16, sem0):
  wid = lax.axis_index("c") * NS + lax.axis_index("s")
  pltpu.sync_copy(perm, pvec)
  for sl in range(SAMPLES_PER_TILE):
    b = wid * SAMPLES_PER_TILE + sl
    p = pvec[pl.ds(b * 16, 16)][0]
    bbase = b * N
    pbase = p * N
    pltpu.sync_copy(sums.at[pl.ds(p * 16, 16)], v16)
    mvec = jnp.full((16,), np.float32(1.0 / N)) * jnp.sum(v16[...])
    lo = jnp.full((16,), np.float32(0.001))
    hi = jnp.full((16,), np.float32(1.0 - 0.001))
    def win(w, carry):
      pltpu.sync_copy(s2k.at[pl.ds(pbase + w * W, W)], xbuf)
      pltpu.sync_copy(s2i.at[pl.ds(bbase + w * W, W)], ibuf)
      def body(i, c):
        v = xbuf[pl.ds(i * 16, 16)]
        g = v - mvec + 0.5
        g = jnp.minimum(jnp.maximum(g, lo), hi)
        xbuf[pl.ds(i * 16, 16)] = g * 0.8 - 0.5
        return c
      lax.fori_loop(0, VPW, body, 0)
      pltpu.async_copy(xbuf, out.at[ibuf], sem0).wait()
      return carry
    lax.fori_loop(0, NWIN, win, 0)


@jax.jit
def kernel(x):
  shape = x.shape
  xf = x.reshape(-1)
  f32, i32 = jnp.float32, jnp.int32
  sort_k = pl.kernel(
      _sort_body,
      out_type=(
          jax.ShapeDtypeStruct((B * 16,), f32),
          jax.ShapeDtypeStruct((B * N,), f32),
          jax.ShapeDtypeStruct((B * N,), i32),
          jax.ShapeDtypeStruct((B * NBINS,), i32),
      ),
      mesh=_mesh(),
      compiler_params=pltpu.CompilerParams(needs_layout_passes=False),
      scratch_types=[
          pltpu.VMEM((NBINS,), i32),
          pltpu.VMEM((NBINS,), i32),
          pltpu.VMEM((W,), f32),
          pltpu.VMEM((W,), f32),
          pltpu.VMEM((W,), i32),
          pltpu.VMEM((W,), i32),
          pltpu.VMEM((16,), f32),
          pltpu.SemaphoreType.DMA,
          pltpu.SemaphoreType.DMA,
      ],
  )
  sums, s1k, s1i, offs2 = sort_k(xf)
  high_k = pl.kernel(
      _highpass_body,
      out_type=(
          jax.ShapeDtypeStruct((B * N,), f32),
          jax.ShapeDtypeStruct((B * N,), i32),
      ),
      mesh=_mesh(),
      compiler_params=pltpu.CompilerParams(needs_layout_passes=False),
      scratch_types=[
          pltpu.VMEM((NBINS,), i32),
          pltpu.VMEM((W,), f32),
          pltpu.VMEM((W,), i32),
          pltpu.VMEM((W,), i32),
          pltpu.SemaphoreType.DMA,
          pltpu.SemaphoreType.DMA,
      ],
  )
  s2k, s2i = high_k(s1k, s1i, offs2)
  match_k = pl.kernel(
      _match_body,
      out_type=jax.ShapeDtypeStruct((B * N,), f32),
      mesh=_mesh(),
      compiler_params=pltpu.CompilerParams(needs_layout_passes=False),
      scratch_types=[
          pltpu.VMEM((W,), f32),
          pltpu.VMEM((W,), i32),
          pltpu.VMEM((B * 16,), i32),
          pltpu.VMEM((16,), f32),
          pltpu.SemaphoreType.DMA,
      ],
  )
  perm16 = jnp.asarray(np.repeat(np.asarray(_PERM, np.int32), 16))
  outf = match_k(s2k, s2i, sums, perm16)
  return outf.reshape(shape)


# speedup vs baseline: 1.2340x; 1.2340x over previous
import functools

import jax
import jax.numpy as jnp
import numpy as np
from jax import lax
from jax.experimental import pallas as pl
from jax.experimental.pallas import tpu as pltpu
from jax.experimental.pallas import tpu_sc as plsc

B = 64
N = 3 * 224 * 224
W = 10752
NWIN = N // W
VPW = W // 16
NBINS = 1 << 15
NC, NS = 2, 16
SAMPLES_PER_TILE = B // (NC * NS)

_PERM = (
    19, 54, 30, 7, 6, 35, 23, 58, 16, 21, 61, 38, 3, 26, 32, 37,
    56, 51, 2, 63, 52, 20, 44, 47, 50, 42, 62, 53, 0, 8, 22, 13,
    29, 34, 18, 24, 1, 48, 5, 45, 49, 33, 55, 60, 57, 27, 10, 15,
    40, 17, 59, 36, 28, 46, 9, 4, 12, 14, 31, 41, 25, 43, 39, 11,
)


def _mesh():
  return plsc.VectorSubcoreMesh(
      core_axis_name="c", subcore_axis_name="s", num_cores=NC, num_subcores=NS)


def _sort_body(x_hbm, sums, s1k, s1i, offs2,
               hist1, hist2, xbuf, kbuf, ibuf, dbuf, v16, sem0, sem1):
  wid = lax.axis_index("c") * NS + lax.axis_index("s")
  lanes = lax.iota(jnp.int32, 16)

  for sl in range(SAMPLES_PER_TILE):
    samp = wid * SAMPLES_PER_TILE + sl
    base = samp * N

    def zero_body(i, carry):
      z = jnp.zeros((16,), jnp.int32)
      hist1[pl.ds(i * 16, 16)] = z
      hist2[pl.ds(i * 16, 16)] = z
      return carry
    lax.fori_loop(0, NBINS // 16, zero_body, 0)

    def histo_win(w, vsum):
      pltpu.sync_copy(x_hbm.at[pl.ds(base + w * W, W)], xbuf)
      def body(i, vs):
        h = xbuf[pl.ds(i * 16, 16)] + 0.5
        k = lax.bitcast_convert_type(h, jnp.int32)
        d1 = k & (NBINS - 1)
        d2 = lax.shift_right_logical(k, 15)
        c1, m1 = plsc.scan_count(d1)
        plsc.addupdate_scatter(hist1, [d1], c1, mask=m1)
        c2, m2 = plsc.scan_count(d2)
        plsc.addupdate_scatter(hist2, [d2], c2, mask=m2)
        return vs + h
      return lax.fori_loop(0, VPW, body, vsum)
    vsum = lax.fori_loop(0, NWIN, histo_win, jnp.zeros((16,), jnp.float32))
    v16[...] = vsum
    pltpu.sync_copy(v16, sums.at[pl.ds(samp * 16, 16)])

    def prefix(hist):
      def body(i, carry):
        v = hist[pl.ds(i * 16, 16)]
        cs = plsc.cumsum(v)
        hist[pl.ds(i * 16, 16)] = carry + (cs - v)
        return carry + jnp.sum(v)
      lax.fori_loop(0, NBINS // 16, body,
                    jnp.full((16,), base, jnp.int32))
    prefix(hist1)
    prefix(hist2)
    pltpu.sync_copy(hist2, offs2.at[pl.ds(samp * NBINS, NBINS)])

    def low_win(w, carry):
      wbase = base + w * W
      pltpu.sync_copy(x_hbm.at[pl.ds(wbase, W)], xbuf)
      def body(i, c):
        h = xbuf[pl.ds(i * 16, 16)] + 0.5
        k = lax.bitcast_convert_type(h, jnp.int32)
        d1 = k & (NBINS - 1)
        cnt, mlast = plsc.scan_count(d1)
        old = plsc.load_gather(hist1, [d1])
        dbuf[pl.ds(i * 16, 16)] = old + cnt - 1
        kbuf[pl.ds(i * 16, 16)] = h
        ibuf[pl.ds(i * 16, 16)] = (wbase + i * 16) + lanes
        plsc.addupdate_scatter(hist1, [d1], cnt, mask=mlast)
        return c
      lax.fori_loop(0, VPW, body, 0)
      cp0 = pltpu.async_copy(kbuf, s1k.at[dbuf], sem0)
      cp1 = pltpu.async_copy(ibuf, s1i.at[dbuf], sem1)
      cp0.wait()
      cp1.wait()
      return carry
    lax.fori_loop(0, NWIN, low_win, 0)


def _highpass_body(s1k, s1i, offs2, s2k, s2i,
                   hist2, xbuf, ibuf, dbuf, sem0, sem1):
  wid = lax.axis_index("c") * NS + lax.axis_index("s")
  for sl in range(SAMPLES_PER_TILE):
    samp = wid * SAMPLES_PER_TILE + sl
    base = samp * N
    pltpu.sync_copy(offs2.at[pl.ds(samp * NBINS, NBINS)], hist2)
    def high_win(w, carry):
      wbase = base + w * W
      pltpu.sync_copy(s1k.at[pl.ds(wbase, W)], xbuf)
      pltpu.sync_copy(s1i.at[pl.ds(wbase, W)], ibuf)
      def body(i, c):
        h = xbuf[pl.ds(i * 16, 16)]
        k = lax.bitcast_convert_type(h, jnp.int32)
        d2 = lax.shift_right_logical(k, 15)
        cnt, mlast = plsc.scan_count(d2)
        old = plsc.load_gather(hist2, [d2])
        dbuf[pl.ds(i * 16, 16)] = old + cnt - 1
        plsc.addupdate_scatter(hist2, [d2], cnt, mask=mlast)
        return c
      lax.fori_loop(0, VPW, body, 0)
      cp0 = pltpu.async_copy(xbuf, s2k.at[dbuf], sem0)
      cp1 = pltpu.async_copy(ibuf, s2i.at[dbuf], sem1)
      cp0.wait()
      cp1.wait()
      return carry
    lax.fori_loop(0, NWIN, high_win, 0)


NCHUNK = 2
Q = N // NCHUNK


def _match_body(s2k, s2i, sums, perm, out, xbuf, ibuf, cbuf, pvec, v16):
  wid = lax.axis_index("c") * NS + lax.axis_index("s")
  pltpu.sync_copy(perm, pvec)
  for sl in range(SAMPLES_PER_TILE):
    b = wid * SAMPLES_PER_TILE + sl
    p = pvec[pl.ds(b * 16, 16)][0]
    bbase = b * N
    pbase = p * N
    pltpu.sync_copy(sums.at[pl.ds(p * 16, 16)], v16)
    mvec = jnp.full((16,), np.float32(1.0 / N)) * jnp.sum(v16[...])
    lo = jnp.full((16,), np.float32(0.001))
    hi = jnp.full((16,), np.float32(1.0 - 0.001))
    for r in range(NCHUNK):
      rbase = bbase + r * Q
      def win(w, carry):
        pltpu.sync_copy(s2k.at[pl.ds(pbase + w * W, W)], xbuf)
        pltpu.sync_copy(s2i.at[pl.ds(bbase + w * W, W)], ibuf)
        def body(i, c):
          v = xbuf[pl.ds(i * 16, 16)]
          g = v - mvec + 0.5
          g = jnp.minimum(jnp.maximum(g, lo), hi)
          g = g * 0.8 - 0.5
          il = ibuf[pl.ds(i * 16, 16)] - rbase
          msk = jnp.logical_and(il >= 0, il < Q)
          plsc.store_scatter(cbuf, [il], g, mask=msk)
          return c
        lax.fori_loop(0, VPW, body, 0)
        return carry
      lax.fori_loop(0, NWIN, win, 0)
      pltpu.sync_copy(cbuf, out.at[pl.ds(rbase, Q)])


@jax.jit
def kernel(x):
  shape = x.shape
  xf = x.reshape(-1)
  f32, i32 = jnp.float32, jnp.int32
  sort_k = pl.kernel(
      _sort_body,
      out_type=(
          jax.ShapeDtypeStruct((B * 16,), f32),
          jax.ShapeDtypeStruct((B * N,), f32),
          jax.ShapeDtypeStruct((B * N,), i32),
          jax.ShapeDtypeStruct((B * NBINS,), i32),
      ),
      mesh=_mesh(),
      compiler_params=pltpu.CompilerParams(needs_layout_passes=False),
      scratch_types=[
          pltpu.VMEM((NBINS,), i32),
          pltpu.VMEM((NBINS,), i32),
          pltpu.VMEM((W,), f32),
          pltpu.VMEM((W,), f32),
          pltpu.VMEM((W,), i32),
          pltpu.VMEM((W,), i32),
          pltpu.VMEM((16,), f32),
          pltpu.SemaphoreType.DMA,
          pltpu.SemaphoreType.DMA,
      ],
  )
  sums, s1k, s1i, offs2 = sort_k(xf)
  high_k = pl.kernel(
      _highpass_body,
      out_type=(
          jax.ShapeDtypeStruct((B * N,), f32),
          jax.ShapeDtypeStruct((B * N,), i32),
      ),
      mesh=_mesh(),
      compiler_params=pltpu.CompilerParams(needs_layout_passes=False),
      scratch_types=[
          pltpu.VMEM((NBINS,), i32),
          pltpu.VMEM((W,), f32),
          pltpu.VMEM((W,), i32),
          pltpu.VMEM((W,), i32),
          pltpu.SemaphoreType.DMA,
          pltpu.SemaphoreType.DMA,
      ],
  )
  s2k, s2i = high_k(s1k, s1i, offs2)
  match_k = pl.kernel(
      _match_body,
      out_type=jax.ShapeDtypeStruct((B * N,), f32),
      mesh=_mesh(),
      compiler_params=pltpu.CompilerParams(needs_layout_passes=False),
      scratch_types=[
          pltpu.VMEM((W,), f32),
          pltpu.VMEM((W,), i32),
          pltpu.VMEM((Q,), f32),
          pltpu.VMEM((B * 16,), i32),
          pltpu.VMEM((16,), f32),
      ],
  )
  perm16 = jnp.asarray(np.repeat(np.asarray(_PERM, np.int32), 16))
  outf = match_k(s2k, s2i, sums, perm16)
  return outf.reshape(shape)
